# baseline (speedup 1.0000x reference)
import functools

import jax
import jax.numpy as jnp
from jax.experimental import pallas as pl
from jax.experimental.pallas import tpu as pltpu

N = 4096
IN_FEAT = 128
BLK1 = 512
BLK3 = 512


def _dot(a, b):
    return jax.lax.dot_general(
        a, b, (((1,), (0,)), ((), ())),
        preferred_element_type=jnp.float32,
    )


def _pass1(adj_ref, b_ref, ab_ref):
    ab_ref[...] = _dot(adj_ref[...], b_ref[...])


def _pass2(ab_ref, w_enc_ref, x_recon_ref):
    m = _dot(w_enc_ref[...], ab_ref[:, IN_FEAT:])
    x_recon_ref[...] = _dot(ab_ref[:, :IN_FEAT], m)


def _pass3(a_ref, w_enc_ref, x_latent_ref):
    x_latent_ref[...] = _dot(a_ref[:, :IN_FEAT], w_enc_ref[...])


@jax.jit
def _run(feat, adj, W_enc, W_dec):
    b = jnp.concatenate([feat, W_dec], axis=1)
    ab = pl.pallas_call(
        _pass1,
        grid=(N // BLK1,),
        in_specs=[
            pl.BlockSpec((BLK1, N), lambda i: (i, 0)),
            pl.BlockSpec((N, 2 * IN_FEAT), lambda i: (0, 0)),
        ],
        out_specs=pl.BlockSpec((BLK1, 2 * IN_FEAT), lambda i: (i, 0)),
        out_shape=jax.ShapeDtypeStruct((N, 2 * IN_FEAT), jnp.float32),
        compiler_params=pltpu.CompilerParams(
            dimension_semantics=("parallel",)),
    )(adj, b)
    x_recon = pl.pallas_call(
        _pass2,
        in_specs=[
            pl.BlockSpec((N, 2 * IN_FEAT), lambda: (0, 0)),
            pl.BlockSpec((IN_FEAT, N), lambda: (0, 0)),
        ],
        out_specs=pl.BlockSpec((N, IN_FEAT), lambda: (0, 0)),
        out_shape=jax.ShapeDtypeStruct((N, IN_FEAT), jnp.float32),
    )(ab, W_enc)
    x_latent = pl.pallas_call(
        _pass3,
        grid=(N // BLK3,),
        in_specs=[
            pl.BlockSpec((BLK3, 2 * IN_FEAT), lambda i: (i, 0)),
            pl.BlockSpec((IN_FEAT, N), lambda i: (0, 0)),
        ],
        out_specs=pl.BlockSpec((BLK3, N), lambda i: (i, 0)),
        out_shape=jax.ShapeDtypeStruct((N, N), jnp.float32),
        compiler_params=pltpu.CompilerParams(
            dimension_semantics=("parallel",)),
    )(ab, W_enc)
    return x_latent, x_recon


def kernel(feat, adj, W_enc, W_dec):
    return _run(feat, adj, W_enc, W_dec)


# speedup vs baseline: 1.0423x; 1.0423x over previous
import functools

import jax
import jax.numpy as jnp
from jax.experimental import pallas as pl
from jax.experimental.pallas import tpu as pltpu

N = 4096
IN_FEAT = 128
BLK = 512
GRID = N // BLK


def _fused_kernel(adj_ref, b_ref, w_enc_ref, x_latent_ref, x_recon_ref, ab_acc):
    i = pl.program_id(0)
    ab = jax.lax.dot_general(
        adj_ref[...], b_ref[...], (((1,), (0,)), ((), ())),
        preferred_element_type=jnp.float32,
        precision=jax.lax.Precision.DEFAULT,
    )
    ab_acc[pl.ds(i * BLK, BLK), :] = ab
    x_latent_ref[...] = jax.lax.dot_general(
        ab[:, :IN_FEAT], w_enc_ref[...], (((1,), (0,)), ((), ())),
        preferred_element_type=jnp.float32,
        precision=jax.lax.Precision.DEFAULT,
    )

    @pl.when(i == GRID - 1)
    def _():
        a = ab_acc[:, :IN_FEAT]
        y = ab_acc[:, IN_FEAT:]
        m = jax.lax.dot_general(
            w_enc_ref[...], y, (((1,), (0,)), ((), ())),
            preferred_element_type=jnp.float32,
            precision=jax.lax.Precision.DEFAULT,
        )
        x_recon_ref[...] = jax.lax.dot_general(
            a, m, (((1,), (0,)), ((), ())),
            preferred_element_type=jnp.float32,
            precision=jax.lax.Precision.DEFAULT,
        )


@jax.jit
def _run(feat, adj, W_enc, W_dec):
    b = jnp.concatenate([feat, W_dec], axis=1)
    x_latent, x_recon = pl.pallas_call(
        _fused_kernel,
        grid=(GRID,),
        in_specs=[
            pl.BlockSpec((BLK, N), lambda i: (i, 0)),
            pl.BlockSpec((N, 2 * IN_FEAT), lambda i: (0, 0)),
            pl.BlockSpec((IN_FEAT, N), lambda i: (0, 0)),
        ],
        out_specs=[
            pl.BlockSpec((BLK, N), lambda i: (i, 0)),
            pl.BlockSpec((N, IN_FEAT), lambda i: (0, 0)),
        ],
        out_shape=[
            jax.ShapeDtypeStruct((N, N), jnp.float32),
            jax.ShapeDtypeStruct((N, IN_FEAT), jnp.float32),
        ],
        scratch_shapes=[pltpu.VMEM((N, 2 * IN_FEAT), jnp.float32)],
    )(adj, b, W_enc)
    return x_latent, x_recon


def kernel(feat, adj, W_enc, W_dec):
    return _run(feat, adj, W_enc, W_dec)


# speedup vs baseline: 1.1034x; 1.0586x over previous
import functools

import jax
import jax.numpy as jnp
from jax.experimental import pallas as pl
from jax.experimental.pallas import tpu as pltpu

N = 4096
IN_FEAT = 128
BLK = 512
GRID = N // BLK


def _dot(a, b):
    return jax.lax.dot_general(
        a, b, (((1,), (0,)), ((), ())),
        preferred_element_type=jnp.float32,
    )


def _kernel(adj_hbm, b_ref, w_enc_ref, x_latent_hbm, x_recon_ref,
            adj_buf, xl_buf, ab_acc, in_sems, out_sems):
    def copy_in(i):
        return pltpu.make_async_copy(
            adj_hbm.at[pl.ds(i * BLK, BLK), :],
            adj_buf.at[i % 2],
            in_sems.at[i % 2],
        )

    def copy_out(i):
        return pltpu.make_async_copy(
            xl_buf.at[i % 2],
            x_latent_hbm.at[pl.ds(i * BLK, BLK), :],
            out_sems.at[i % 2],
        )

    copy_in(0).start()
    for i in range(GRID):
        if i + 1 < GRID:
            copy_in(i + 1).start()
        copy_in(i).wait()
        ab = _dot(adj_buf[i % 2], b_ref[...])
        ab_acc[pl.ds(i * BLK, BLK), :] = ab
        if i >= 2:
            copy_out(i - 2).wait()
        xl_buf[i % 2] = _dot(ab[:, :IN_FEAT], w_enc_ref[...])
        copy_out(i).start()

    a = ab_acc[:, :IN_FEAT]
    y = ab_acc[:, IN_FEAT:]
    m = _dot(w_enc_ref[...], y)
    x_recon_ref[...] = _dot(a, m)
    copy_out(GRID - 2).wait()
    copy_out(GRID - 1).wait()


@jax.jit
def _run(feat, adj, W_enc, W_dec):
    b = jnp.concatenate([feat, W_dec], axis=1)
    x_latent, x_recon = pl.pallas_call(
        _kernel,
        in_specs=[
            pl.BlockSpec(memory_space=pltpu.MemorySpace.HBM),
            pl.BlockSpec((N, 2 * IN_FEAT), lambda: (0, 0)),
            pl.BlockSpec((IN_FEAT, N), lambda: (0, 0)),
        ],
        out_specs=[
            pl.BlockSpec(memory_space=pltpu.MemorySpace.HBM),
            pl.BlockSpec((N, IN_FEAT), lambda: (0, 0)),
        ],
        out_shape=[
            jax.ShapeDtypeStruct((N, N), jnp.float32),
            jax.ShapeDtypeStruct((N, IN_FEAT), jnp.float32),
        ],
        scratch_shapes=[
            pltpu.VMEM((2, BLK, N), jnp.float32),
            pltpu.VMEM((2, BLK, N), jnp.float32),
            pltpu.VMEM((N, 2 * IN_FEAT), jnp.float32),
            pltpu.SemaphoreType.DMA((2,)),
            pltpu.SemaphoreType.DMA((2,)),
        ],
    )(adj, b, W_enc)
    return x_latent, x_recon


def kernel(feat, adj, W_enc, W_dec):
    return _run(feat, adj, W_enc, W_dec)


# speedup vs baseline: 1.1084x; 1.0046x over previous
import functools

import jax
import jax.numpy as jnp
from jax.experimental import pallas as pl
from jax.experimental.pallas import tpu as pltpu

N = 4096
IN_FEAT = 128
BLK = 512
GRID = N // BLK


def _dot(a, b):
    return jax.lax.dot_general(
        a, b, (((1,), (0,)), ((), ())),
        preferred_element_type=jnp.float32,
    )


def _kernel(adj_hbm, b_ref, w_enc_ref, x_latent_hbm, x_recon_ref,
            adj_buf, xl_buf, ab_acc, in_sems, out_sems):
    def copy_in(i):
        return pltpu.make_async_copy(
            adj_hbm.at[pl.ds(i * BLK, BLK), :],
            adj_buf.at[i % 3],
            in_sems.at[i % 3],
        )

    def copy_out(i):
        return pltpu.make_async_copy(
            xl_buf.at[i % 2],
            x_latent_hbm.at[pl.ds(i * BLK, BLK), :],
            out_sems.at[i % 2],
        )

    copy_in(0).start()
    copy_in(1).start()
    for i in range(GRID):
        if i + 2 < GRID:
            copy_in(i + 2).start()
        copy_in(i).wait()
        ab = _dot(adj_buf[i % 3], b_ref[...])
        ab_acc[pl.ds(i * BLK, BLK), :] = ab
        if i >= 2:
            copy_out(i - 2).wait()
        xl_buf[i % 2] = _dot(ab[:, :IN_FEAT], w_enc_ref[...])
        copy_out(i).start()

    a = ab_acc[:, :IN_FEAT]
    y = ab_acc[:, IN_FEAT:]
    m = _dot(w_enc_ref[...].astype(jnp.float32), y)
    x_recon_ref[...] = _dot(a, m)
    copy_out(GRID - 2).wait()
    copy_out(GRID - 1).wait()


@jax.jit
def _run(feat, adj, W_enc, W_dec):
    b = jnp.concatenate([feat, W_dec], axis=1).astype(jnp.bfloat16)
    x_latent, x_recon = pl.pallas_call(
        _kernel,
        in_specs=[
            pl.BlockSpec(memory_space=pltpu.MemorySpace.HBM),
            pl.BlockSpec((N, 2 * IN_FEAT), lambda: (0, 0)),
            pl.BlockSpec((IN_FEAT, N), lambda: (0, 0)),
        ],
        out_specs=[
            pl.BlockSpec(memory_space=pltpu.MemorySpace.HBM),
            pl.BlockSpec((N, IN_FEAT), lambda: (0, 0)),
        ],
        out_shape=[
            jax.ShapeDtypeStruct((N, N), jnp.float32),
            jax.ShapeDtypeStruct((N, IN_FEAT), jnp.float32),
        ],
        scratch_shapes=[
            pltpu.VMEM((3, BLK, N), jnp.float32),
            pltpu.VMEM((2, BLK, N), jnp.float32),
            pltpu.VMEM((N, 2 * IN_FEAT), jnp.float32),
            pltpu.SemaphoreType.DMA((3,)),
            pltpu.SemaphoreType.DMA((2,)),
        ],
    )(adj, b, W_enc.astype(jnp.bfloat16))
    return x_latent, x_recon


def kernel(feat, adj, W_enc, W_dec):
    return _run(feat, adj, W_enc, W_dec)
